# baseline (speedup 1.0000x reference)
import jax
import jax.numpy as jnp
from jax.experimental import pallas as pl
from jax.experimental.pallas import tpu as pltpu


def _copy_kernel(src, dst):
    pltpu.sync_copy(src, dst)


def kernel(embedding):
    return pl.pallas_call(
        _copy_kernel,
        in_specs=[pl.BlockSpec(memory_space=pl.ANY)],
        out_specs=pl.BlockSpec(memory_space=pl.ANY),
        out_shape=jax.ShapeDtypeStruct(embedding.shape, embedding.dtype),
    )(embedding)


# speedup vs baseline: 3.7175x; 3.7175x over previous
import jax
import jax.numpy as jnp
from jax import lax
from jax.experimental import pallas as pl
from jax.experimental.pallas import tpu as pltpu
from jax.experimental.pallas import tpu_sc as plsc

_LANES = 128
_ROWS = 1000 * 77 * 512 // _LANES
_GROUPS = _ROWS // 8
_NC, _NS = 2, 16
_NW = _NC * _NS
_WGROUPS = 1204
_NCHUNK = 28
_CROWS = _WGROUPS * 8 // _NCHUNK


def _copy_body(src, out, buf, sems):
    wid = lax.axis_index("s") * _NC + lax.axis_index("c")
    base = pl.multiple_of((wid * _GROUPS // _NW) * 8, 8)

    def load(g):
        return pltpu.make_async_copy(
            src.at[pl.ds(base + g * _CROWS, _CROWS)], buf.at[g % 2], sems.at[g % 2]
        )

    def store(g):
        return pltpu.make_async_copy(
            buf.at[g % 2], out.at[pl.ds(base + g * _CROWS, _CROWS)], sems.at[2 + g % 2]
        )

    load(0).start()
    load(1).start()
    for g in range(_NCHUNK):
        load(g).wait()
        store(g).start()
        if g + 2 < _NCHUNK:
            store(g).wait()
            load(g + 2).start()
    store(_NCHUNK - 2).wait()
    store(_NCHUNK - 1).wait()


@jax.jit
def _sc_copy(flat):
    mesh = plsc.VectorSubcoreMesh(core_axis_name="c", subcore_axis_name="s")
    return pl.kernel(
        _copy_body,
        out_type=jax.ShapeDtypeStruct((_ROWS, _LANES), jnp.float32),
        mesh=mesh,
        scratch_types=[
            pltpu.VMEM((2, _CROWS, _LANES), jnp.float32),
            pltpu.SemaphoreType.DMA((4,)),
        ],
    )(flat)


def kernel(embedding):
    flat = embedding.reshape(_ROWS, _LANES)
    return _sc_copy(flat).reshape(embedding.shape)


# speedup vs baseline: 15.3833x; 4.1380x over previous
import jax
import jax.numpy as jnp
from jax import lax
from jax.experimental import pallas as pl
from jax.experimental.pallas import tpu as pltpu
from jax.experimental.pallas import tpu_sc as plsc

_N, _CTX, _D = 1000, 77, 512
_NC, _NS = 2, 16
_NW = _NC * _NS
_WROWS = 32


def _copy_body(src, out, buf, sems):
    wid = lax.axis_index("s") * _NC + lax.axis_index("c")
    base = wid * _N // _NW

    def load(g):
        return pltpu.make_async_copy(src.at[base + g], buf.at[g % 2], sems.at[g % 2])

    def store(g):
        return pltpu.make_async_copy(buf.at[g % 2], out.at[base + g], sems.at[2 + g % 2])

    load(0).start()
    load(1).start()
    for g in range(_WROWS):
        load(g).wait()
        store(g).start()
        if g + 2 < _WROWS:
            store(g).wait()
            load(g + 2).start()
    store(_WROWS - 2).wait()
    store(_WROWS - 1).wait()


@jax.jit
def _sc_copy(embedding):
    mesh = plsc.VectorSubcoreMesh(core_axis_name="c", subcore_axis_name="s")
    return pl.kernel(
        _copy_body,
        out_type=jax.ShapeDtypeStruct((_N, _CTX, _D), jnp.float32),
        mesh=mesh,
        scratch_types=[
            pltpu.VMEM((2, _CTX, _D), jnp.float32),
            pltpu.SemaphoreType.DMA((4,)),
        ],
        compiler_params=pltpu.CompilerParams(use_tc_tiling_on_sc=True),
    )(embedding)


def kernel(embedding):
    return _sc_copy(embedding)


# speedup vs baseline: 39.4743x; 2.5660x over previous
import jax
import jax.numpy as jnp
from jax import lax
from jax.experimental import pallas as pl
from jax.experimental.pallas import tpu as pltpu
from jax.experimental.pallas import tpu_sc as plsc

_ROWS, _COLS = 77000, 512
_GROUPS = _ROWS // 8
_NC, _NS = 2, 16
_NW = _NC * _NS
_WGROUPS = 304
_NCHUNK = 38
_CROWS = _WGROUPS * 8 // _NCHUNK


def _copy_body(src, out, buf, sems):
    wid = lax.axis_index("s") * _NC + lax.axis_index("c")
    base_g = jnp.minimum(wid * _GROUPS // _NW, _GROUPS - _WGROUPS)
    base = pl.multiple_of(base_g * 8, 8)

    def load(g):
        return pltpu.make_async_copy(
            src.at[pl.ds(base + g * _CROWS, _CROWS)], buf.at[g % 2], sems.at[g % 2]
        )

    def store(g):
        return pltpu.make_async_copy(
            buf.at[g % 2], out.at[pl.ds(base + g * _CROWS, _CROWS)], sems.at[2 + g % 2]
        )

    load(0).start()
    load(1).start()
    for g in range(_NCHUNK):
        load(g).wait()
        store(g).start()
        if g + 2 < _NCHUNK:
            store(g).wait()
            load(g + 2).start()
    store(_NCHUNK - 2).wait()
    store(_NCHUNK - 1).wait()


@jax.jit
def _sc_copy(flat):
    mesh = plsc.VectorSubcoreMesh(core_axis_name="c", subcore_axis_name="s")
    return pl.kernel(
        _copy_body,
        out_type=jax.ShapeDtypeStruct((_ROWS, _COLS), jnp.float32),
        mesh=mesh,
        scratch_types=[
            pltpu.VMEM((2, _CROWS, _COLS), jnp.float32),
            pltpu.SemaphoreType.DMA((4,)),
        ],
        compiler_params=pltpu.CompilerParams(use_tc_tiling_on_sc=True),
    )(flat)


def kernel(embedding):
    flat = jnp.transpose(embedding, (1, 0, 2)).reshape(_ROWS, _COLS)
    out = _sc_copy(flat)
    return jnp.transpose(out.reshape(77, 1000, 512), (1, 0, 2))
